# Initial kernel scaffold; baseline (speedup 1.0000x reference)
import jax
import jax.numpy as jnp
from jax.experimental import pallas as pl


def kernel(xyz, features, W1, b1, g1, beta1, Wd1, bd1, Wd2, bd2, Wq, Wk, Wv, W2, b2, g2, beta2):
    raise NotImplementedError("write your pallas kernel here")


# speedup vs baseline: 14.8257x; 14.8257x over previous
import functools

import jax
import jax.numpy as jnp
import numpy as np
from jax import lax
from jax.experimental import pallas as pl
from jax.experimental.pallas import tpu as pltpu
from jax.experimental.pallas import tpu_sc as plsc

_B, _N, _DP, _DM, _K = 4, 2048, 32, 64, 16
_M = _B * _N
_TN = 256
_TC = 256
_TW = 128
_EPS = 1e-5


def _ln(x, g, b):
    mu = jnp.mean(x, axis=-1, keepdims=True)
    var = jnp.mean((x - mu) ** 2, axis=-1, keepdims=True)
    return (x - mu) * lax.rsqrt(var + _EPS) * g + b


def _proj_body(xyz_ref, f_ref, W1_ref, b1_ref, g1_ref, beta1_ref,
               Wq_ref, Wk_ref, Wv_ref,
               q_ref, kf_ref, x16_ref, tab_ref):
    xyz3 = xyz_ref[0]
    f = f_ref[0]
    x = _ln(jnp.dot(f, W1_ref[...], preferred_element_type=jnp.float32)
            + b1_ref[...], g1_ref[...], beta1_ref[...])
    q_ref[0] = jnp.dot(x, Wq_ref[...], preferred_element_type=jnp.float32)
    kf_ref[0] = jnp.dot(x, Wk_ref[...], preferred_element_type=jnp.float32)
    vf = jnp.dot(x, Wv_ref[...], preferred_element_type=jnp.float32)
    sq = jnp.sum(xyz3 * xyz3, axis=-1, keepdims=True)
    x16 = jnp.concatenate(
        [xyz3, sq, jnp.zeros((_N, 12), jnp.float32)], axis=1)
    x16_ref[0] = x16
    tab_ref[0] = jnp.concatenate(
        [vf, x16, jnp.zeros((_N, _TW - _DM - 16), jnp.float32)], axis=1)


def _proj_call(xyz, features, W1, b1r, g1r, beta1r, Wq, Wk, Wv, interpret=False):
    full = lambda b: (b, 0, 0)
    w2d = lambda b: (0, 0)
    return pl.pallas_call(
        _proj_body,
        grid=(_B,),
        in_specs=[
            pl.BlockSpec((1, _N, 3), full),
            pl.BlockSpec((1, _N, _DP), full),
            pl.BlockSpec((_DP, _DM), w2d),
            pl.BlockSpec((1, _DM), w2d),
            pl.BlockSpec((1, _DM), w2d),
            pl.BlockSpec((1, _DM), w2d),
            pl.BlockSpec((_DM, _DM), w2d),
            pl.BlockSpec((_DM, _DM), w2d),
            pl.BlockSpec((_DM, _DM), w2d),
        ],
        out_specs=[
            pl.BlockSpec((1, _N, _DM), full),
            pl.BlockSpec((1, _N, _DM), full),
            pl.BlockSpec((1, _N, 16), full),
            pl.BlockSpec((1, _N, _TW), full),
        ],
        out_shape=[
            jax.ShapeDtypeStruct((_B, _N, _DM), jnp.float32),
            jax.ShapeDtypeStruct((_B, _N, _DM), jnp.float32),
            jax.ShapeDtypeStruct((_B, _N, 16), jnp.float32),
            jax.ShapeDtypeStruct((_B, _N, _TW), jnp.float32),
        ],
        interpret=interpret,
    )(xyz, features, W1, b1r, g1r, beta1r, Wq, Wk, Wv)


def _topk_body(xt_ref, xa_ref, sqt_ref, kf_ref, q_ref, gidx_ref, qks_ref):
    b = pl.program_id(0)
    tile = xt_ref[0]
    xa = xa_ref[0]
    li = lax.broadcasted_iota(jnp.int32, (1, 16), 1)
    m3 = (li < 3).astype(jnp.float32)
    pd = lax.dot_general(tile * m3, xa * m3, (((1,), (1,)), ((), ())),
                         preferred_element_type=jnp.float32)
    dist = (tile[:, 3:4] + sqt_ref[0]) - 2.0 * pd
    qk = lax.dot_general(q_ref[0], kf_ref[0], (((1,), (1,)), ((), ())),
                         preferred_element_type=jnp.float32)

    iota_n = lax.broadcasted_iota(jnp.int32, (_TN, _N), 1)
    iota_k = lax.broadcasted_iota(jnp.int32, (_TN, _K), 1)
    idx_acc = jnp.zeros((_TN, _K), jnp.int32)
    qk_acc = jnp.zeros((_TN, _K), jnp.float32)
    d = dist
    for t in range(_K):
        m = jnp.min(d, axis=1, keepdims=True)
        idx = jnp.min(jnp.where(d <= m, iota_n, _N), axis=1, keepdims=True)
        onehot = iota_n == idx
        qk_t = jnp.sum(jnp.where(onehot, qk, 0.0), axis=1, keepdims=True)
        d = jnp.where(onehot, jnp.float32(np.inf), d)
        sel = iota_k == t
        idx_acc = jnp.where(sel, idx, idx_acc)
        qk_acc = jnp.where(sel, qk_t, qk_acc)
    gidx_ref[0] = idx_acc + b * _N
    qks_ref[0] = qk_acc


def _topk_call(x16, kf, q, interpret=False):
    sqt = x16[:, :, 3].reshape(_B, 1, _N)
    return pl.pallas_call(
        _topk_body,
        grid=(_B, _N // _TN),
        in_specs=[
            pl.BlockSpec((1, _TN, 16), lambda b, j: (b, j, 0)),
            pl.BlockSpec((1, _N, 16), lambda b, j: (b, 0, 0)),
            pl.BlockSpec((1, 1, _N), lambda b, j: (b, 0, 0)),
            pl.BlockSpec((1, _N, _DM), lambda b, j: (b, 0, 0)),
            pl.BlockSpec((1, _TN, _DM), lambda b, j: (b, j, 0)),
        ],
        out_specs=[
            pl.BlockSpec((1, _TN, _K), lambda b, j: (b, j, 0)),
            pl.BlockSpec((1, _TN, _K), lambda b, j: (b, j, 0)),
        ],
        out_shape=[
            jax.ShapeDtypeStruct((_B, _N, _K), jnp.int32),
            jax.ShapeDtypeStruct((_B, _N, _K), jnp.float32),
        ],
        interpret=interpret,
    )(x16, x16, sqt, kf, q)


def _gather_call(tab, gidx):
    info = plsc.get_sparse_core_info()
    nw = info.num_cores * info.num_subcores
    tot = _M * _K
    per_w = tot // nw
    ch = 128
    n_ch = per_w // ch
    mesh = plsc.VectorSubcoreMesh(core_axis_name="c", subcore_axis_name="s")

    @functools.partial(
        pl.kernel,
        mesh=mesh,
        out_type=jax.ShapeDtypeStruct((tot, _TW), jnp.float32),
        scratch_types=[
            pltpu.VMEM((ch,), jnp.int32),
            pltpu.VMEM((ch, _TW), jnp.float32),
            pltpu.SemaphoreType.DMA,
        ],
    )
    def _sc_gather(tab_hbm, idx_hbm, out_hbm, idx_v, rows_v, sem):
        wid = lax.axis_index("s") * info.num_cores + lax.axis_index("c")
        base = wid * per_w

        def body(c, carry):
            off = base + c * ch
            pltpu.sync_copy(idx_hbm.at[pl.ds(off, ch)], idx_v)
            pltpu.async_copy(tab_hbm.at[idx_v], rows_v, sem).wait()
            pltpu.sync_copy(rows_v, out_hbm.at[pl.ds(off, ch)])
            return carry

        lax.fori_loop(0, n_ch, body, 0)

    return _sc_gather(tab, gidx)


def _attn_body(g_ref, q_ref, qks_ref, x16_ref, f_ref,
               Wd1p_ref, bd1_ref, Wd2_ref, bd2_ref,
               W2_ref, b2_ref, g2_ref, beta2_ref,
               res_ref, attn_ref):
    g = g_ref[...]
    vsel = g[:, :, :_DM]
    xs = g[:, :, _DM:_DM + 16]
    xi = x16_ref[...]
    pos = xi[:, None, :] - xs
    posf = pos.reshape(_TC * _K, 16)
    h = jnp.maximum(
        jnp.dot(posf, Wd1p_ref[...], preferred_element_type=jnp.float32)
        + bd1_ref[...], 0.0)
    pe = (jnp.dot(h, Wd2_ref[...], preferred_element_type=jnp.float32)
          + bd2_ref[...]).reshape(_TC, _K, _DM)
    q = q_ref[...]
    s = (qks_ref[...] + jnp.sum(q[:, None, :] * pe, axis=-1)) \
        * jnp.float32(1.0 / np.sqrt(_DM))
    m = jnp.max(s, axis=-1, keepdims=True)
    e = jnp.exp(s - m)
    a = e / jnp.sum(e, axis=-1, keepdims=True)
    attn_ref[...] = a
    outv = jnp.sum(a[:, :, None] * (vsel + pe), axis=1)
    y = jnp.dot(outv, W2_ref[...], preferred_element_type=jnp.float32) \
        + b2_ref[...]
    res_ref[...] = _ln(y, g2_ref[...], beta2_ref[...]) + f_ref[...]


def _attn_call(g3, qf, qks, x16f, ff, Wd1p, bd1r, Wd2, bd2r, W2, b2r, g2r,
               beta2r, interpret=False):
    row = lambda i: (i, 0)
    row3 = lambda i: (i, 0, 0)
    w2d = lambda i: (0, 0)
    return pl.pallas_call(
        _attn_body,
        grid=(_M // _TC,),
        in_specs=[
            pl.BlockSpec((_TC, _K, _TW), row3),
            pl.BlockSpec((_TC, _DM), row),
            pl.BlockSpec((_TC, _K), row),
            pl.BlockSpec((_TC, 16), row),
            pl.BlockSpec((_TC, _DP), row),
            pl.BlockSpec((16, _DM), w2d),
            pl.BlockSpec((1, _DM), w2d),
            pl.BlockSpec((_DM, _DM), w2d),
            pl.BlockSpec((1, _DM), w2d),
            pl.BlockSpec((_DM, _DP), w2d),
            pl.BlockSpec((1, _DP), w2d),
            pl.BlockSpec((1, _DP), w2d),
            pl.BlockSpec((1, _DP), w2d),
        ],
        out_specs=[
            pl.BlockSpec((_TC, _DP), row),
            pl.BlockSpec((_TC, _K), row),
        ],
        out_shape=[
            jax.ShapeDtypeStruct((_M, _DP), jnp.float32),
            jax.ShapeDtypeStruct((_M, _K), jnp.float32),
        ],
        interpret=interpret,
    )(g3, qf, qks, x16f, ff, Wd1p, bd1r, Wd2, bd2r, W2, b2r, g2r, beta2r)


def kernel(xyz, features, W1, b1, g1, beta1, Wd1, bd1, Wd2, bd2,
           Wq, Wk, Wv, W2, b2, g2, beta2):
    b1r = b1.reshape(1, _DM)
    g1r = g1.reshape(1, _DM)
    beta1r = beta1.reshape(1, _DM)
    bd1r = bd1.reshape(1, _DM)
    bd2r = bd2.reshape(1, _DM)
    b2r = b2.reshape(1, _DP)
    g2r = g2.reshape(1, _DP)
    beta2r = beta2.reshape(1, _DP)
    Wd1p = jnp.zeros((16, _DM), jnp.float32).at[:3].set(Wd1)

    q, kf, x16, tab = _proj_call(xyz, features, W1, b1r, g1r, beta1r,
                                 Wq, Wk, Wv)
    gidx, qks = _topk_call(x16, kf, q)
    gathered = _gather_call(tab.reshape(_M, _TW), gidx.reshape(_M * _K))
    g3 = gathered.reshape(_M, _K, _TW)
    res, attn = _attn_call(g3, q.reshape(_M, _DM), qks.reshape(_M, _K),
                           x16.reshape(_M, 16), features.reshape(_M, _DP),
                           Wd1p, bd1r, Wd2, bd2r, W2, b2r, g2r, beta2r)
    return res.reshape(_B, _N, _DP), attn.reshape(_B, _N, 1, _K)


# speedup vs baseline: 16.0691x; 1.0839x over previous
import functools

import jax
import jax.numpy as jnp
import numpy as np
from jax import lax
from jax.experimental import pallas as pl
from jax.experimental.pallas import tpu as pltpu
from jax.experimental.pallas import tpu_sc as plsc

_B, _N, _DP, _DM, _K = 4, 2048, 32, 64, 16
_M = _B * _N
_TN = 256
_TC = 256
_TW = 128
_EPS = 1e-5


def _ln(x, g, b):
    mu = jnp.mean(x, axis=-1, keepdims=True)
    var = jnp.mean((x - mu) ** 2, axis=-1, keepdims=True)
    return (x - mu) * lax.rsqrt(var + _EPS) * g + b


def _proj_body(xyz_ref, f_ref, W1_ref, b1_ref, g1_ref, beta1_ref,
               Wq_ref, Wk_ref, Wv_ref,
               q_ref, kf_ref, x16_ref, tab_ref):
    xyz3 = xyz_ref[0]
    f = f_ref[0]
    x = _ln(jnp.dot(f, W1_ref[...], preferred_element_type=jnp.float32)
            + b1_ref[...], g1_ref[...], beta1_ref[...])
    q_ref[0] = jnp.dot(x, Wq_ref[...], preferred_element_type=jnp.float32)
    kf_ref[0] = jnp.dot(x, Wk_ref[...], preferred_element_type=jnp.float32)
    vf = jnp.dot(x, Wv_ref[...], preferred_element_type=jnp.float32)
    sq = jnp.sum(xyz3 * xyz3, axis=-1, keepdims=True)
    x16 = jnp.concatenate(
        [xyz3, sq, jnp.zeros((_N, 12), jnp.float32)], axis=1)
    x16_ref[0] = x16
    tab_ref[0] = jnp.concatenate(
        [vf, x16, jnp.zeros((_N, _TW - _DM - 16), jnp.float32)], axis=1)


def _proj_call(xyz, features, W1, b1r, g1r, beta1r, Wq, Wk, Wv, interpret=False):
    full = lambda b: (b, 0, 0)
    w2d = lambda b: (0, 0)
    return pl.pallas_call(
        _proj_body,
        grid=(_B,),
        in_specs=[
            pl.BlockSpec((1, _N, 3), full),
            pl.BlockSpec((1, _N, _DP), full),
            pl.BlockSpec((_DP, _DM), w2d),
            pl.BlockSpec((1, _DM), w2d),
            pl.BlockSpec((1, _DM), w2d),
            pl.BlockSpec((1, _DM), w2d),
            pl.BlockSpec((_DM, _DM), w2d),
            pl.BlockSpec((_DM, _DM), w2d),
            pl.BlockSpec((_DM, _DM), w2d),
        ],
        out_specs=[
            pl.BlockSpec((1, _N, _DM), full),
            pl.BlockSpec((1, _N, _DM), full),
            pl.BlockSpec((1, _N, 16), full),
            pl.BlockSpec((1, _N, _TW), full),
        ],
        out_shape=[
            jax.ShapeDtypeStruct((_B, _N, _DM), jnp.float32),
            jax.ShapeDtypeStruct((_B, _N, _DM), jnp.float32),
            jax.ShapeDtypeStruct((_B, _N, 16), jnp.float32),
            jax.ShapeDtypeStruct((_B, _N, _TW), jnp.float32),
        ],
        compiler_params=pltpu.CompilerParams(
            dimension_semantics=("parallel",)),
        interpret=interpret,
    )(xyz, features, W1, b1r, g1r, beta1r, Wq, Wk, Wv)


def _topk_body(xt_ref, xa_ref, sqt_ref, kf_ref, q_ref, gidx_ref, qks_ref):
    b = pl.program_id(0)
    tile = xt_ref[0]
    xa = xa_ref[0]
    li = lax.broadcasted_iota(jnp.int32, (1, 16), 1)
    m3 = (li < 3).astype(jnp.float32)
    pd = lax.dot_general(tile * m3, xa * m3, (((1,), (1,)), ((), ())),
                         preferred_element_type=jnp.float32)
    dist = (tile[:, 3:4] + sqt_ref[0]) - 2.0 * pd
    qk = lax.dot_general(q_ref[0], kf_ref[0], (((1,), (1,)), ((), ())),
                         preferred_element_type=jnp.float32)

    iota_f = lax.broadcasted_iota(jnp.int32, (_TN, _N), 1).astype(jnp.float32)
    iota_k = lax.broadcasted_iota(jnp.int32, (_TN, _K), 1)
    idx_acc = jnp.zeros((_TN, _K), jnp.float32)
    qk_acc = jnp.zeros((_TN, _K), jnp.float32)
    d = dist
    for t in range(_K):
        m = jnp.min(d, axis=1, keepdims=True)
        cand = jnp.where(d <= m, iota_f, jnp.float32(_N))
        idx = jnp.min(cand, axis=1, keepdims=True)
        onehot = cand == idx
        qk_t = jnp.sum(jnp.where(onehot, qk, 0.0), axis=1, keepdims=True)
        d = jnp.where(onehot, jnp.float32(np.inf), d)
        sel = iota_k == t
        idx_acc = jnp.where(sel, idx, idx_acc)
        qk_acc = jnp.where(sel, qk_t, qk_acc)
    gidx_ref[0] = idx_acc.astype(jnp.int32) + b * _N
    qks_ref[0] = qk_acc


def _topk_call(x16, kf, q, interpret=False):
    sqt = x16[:, :, 3].reshape(_B, 1, _N)
    return pl.pallas_call(
        _topk_body,
        grid=(_B, _N // _TN),
        in_specs=[
            pl.BlockSpec((1, _TN, 16), lambda b, j: (b, j, 0)),
            pl.BlockSpec((1, _N, 16), lambda b, j: (b, 0, 0)),
            pl.BlockSpec((1, 1, _N), lambda b, j: (b, 0, 0)),
            pl.BlockSpec((1, _N, _DM), lambda b, j: (b, 0, 0)),
            pl.BlockSpec((1, _TN, _DM), lambda b, j: (b, j, 0)),
        ],
        out_specs=[
            pl.BlockSpec((1, _TN, _K), lambda b, j: (b, j, 0)),
            pl.BlockSpec((1, _TN, _K), lambda b, j: (b, j, 0)),
        ],
        out_shape=[
            jax.ShapeDtypeStruct((_B, _N, _K), jnp.int32),
            jax.ShapeDtypeStruct((_B, _N, _K), jnp.float32),
        ],
        compiler_params=pltpu.CompilerParams(
            dimension_semantics=("parallel", "parallel")),
        interpret=interpret,
    )(x16, x16, sqt, kf, q)


def _gather_call(tab, gidx):
    info = plsc.get_sparse_core_info()
    nw = info.num_cores * info.num_subcores
    tot = _M * _K
    per_w = tot // nw
    ch = 128
    n_ch = per_w // ch
    mesh = plsc.VectorSubcoreMesh(core_axis_name="c", subcore_axis_name="s")

    @functools.partial(
        pl.kernel,
        mesh=mesh,
        out_type=jax.ShapeDtypeStruct((tot, _TW), jnp.float32),
        scratch_types=[
            pltpu.VMEM((ch,), jnp.int32),
            pltpu.VMEM((ch, _TW), jnp.float32),
            pltpu.SemaphoreType.DMA,
        ],
    )
    def _sc_gather(tab_hbm, idx_hbm, out_hbm, idx_v, rows_v, sem):
        wid = lax.axis_index("s") * info.num_cores + lax.axis_index("c")
        base = wid * per_w

        def body(c, carry):
            off = base + c * ch
            pltpu.sync_copy(idx_hbm.at[pl.ds(off, ch)], idx_v)
            pltpu.async_copy(tab_hbm.at[idx_v], rows_v, sem).wait()
            pltpu.sync_copy(rows_v, out_hbm.at[pl.ds(off, ch)])
            return carry

        lax.fori_loop(0, n_ch, body, 0)

    return _sc_gather(tab, gidx)


def _attn_body(g_ref, q_ref, qks_ref, x16_ref, f_ref,
               Wd1p_ref, bd1_ref, Wd2_ref, bd2_ref,
               W2_ref, b2_ref, g2_ref, beta2_ref,
               res_ref, attn_ref):
    g = g_ref[...]
    vsel = g[:, :, :_DM]
    xs = g[:, :, _DM:_DM + 16]
    xi = x16_ref[...]
    pos = xi[:, None, :] - xs
    posf = pos.reshape(_TC * _K, 16)
    h = jnp.maximum(
        jnp.dot(posf, Wd1p_ref[...], preferred_element_type=jnp.float32)
        + bd1_ref[...], 0.0)
    pe = (jnp.dot(h, Wd2_ref[...], preferred_element_type=jnp.float32)
          + bd2_ref[...]).reshape(_TC, _K, _DM)
    q = q_ref[...]
    s = (qks_ref[...] + jnp.sum(q[:, None, :] * pe, axis=-1)) \
        * jnp.float32(1.0 / np.sqrt(_DM))
    m = jnp.max(s, axis=-1, keepdims=True)
    e = jnp.exp(s - m)
    a = e / jnp.sum(e, axis=-1, keepdims=True)
    attn_ref[...] = a
    outv = jnp.sum(a[:, :, None] * (vsel + pe), axis=1)
    y = jnp.dot(outv, W2_ref[...], preferred_element_type=jnp.float32) \
        + b2_ref[...]
    res_ref[...] = _ln(y, g2_ref[...], beta2_ref[...]) + f_ref[...]


def _attn_call(g3, qf, qks, x16f, ff, Wd1p, bd1r, Wd2, bd2r, W2, b2r, g2r,
               beta2r, interpret=False):
    row = lambda i: (i, 0)
    row3 = lambda i: (i, 0, 0)
    w2d = lambda i: (0, 0)
    return pl.pallas_call(
        _attn_body,
        grid=(_M // _TC,),
        in_specs=[
            pl.BlockSpec((_TC, _K, _TW), row3),
            pl.BlockSpec((_TC, _DM), row),
            pl.BlockSpec((_TC, _K), row),
            pl.BlockSpec((_TC, 16), row),
            pl.BlockSpec((_TC, _DP), row),
            pl.BlockSpec((16, _DM), w2d),
            pl.BlockSpec((1, _DM), w2d),
            pl.BlockSpec((_DM, _DM), w2d),
            pl.BlockSpec((1, _DM), w2d),
            pl.BlockSpec((_DM, _DP), w2d),
            pl.BlockSpec((1, _DP), w2d),
            pl.BlockSpec((1, _DP), w2d),
            pl.BlockSpec((1, _DP), w2d),
        ],
        out_specs=[
            pl.BlockSpec((_TC, _DP), row),
            pl.BlockSpec((_TC, _K), row),
        ],
        out_shape=[
            jax.ShapeDtypeStruct((_M, _DP), jnp.float32),
            jax.ShapeDtypeStruct((_M, _K), jnp.float32),
        ],
        compiler_params=pltpu.CompilerParams(
            dimension_semantics=("parallel",)),
        interpret=interpret,
    )(g3, qf, qks, x16f, ff, Wd1p, bd1r, Wd2, bd2r, W2, b2r, g2r, beta2r)


def kernel(xyz, features, W1, b1, g1, beta1, Wd1, bd1, Wd2, bd2,
           Wq, Wk, Wv, W2, b2, g2, beta2):
    b1r = b1.reshape(1, _DM)
    g1r = g1.reshape(1, _DM)
    beta1r = beta1.reshape(1, _DM)
    bd1r = bd1.reshape(1, _DM)
    bd2r = bd2.reshape(1, _DM)
    b2r = b2.reshape(1, _DP)
    g2r = g2.reshape(1, _DP)
    beta2r = beta2.reshape(1, _DP)
    Wd1p = jnp.zeros((16, _DM), jnp.float32).at[:3].set(Wd1)

    q, kf, x16, tab = _proj_call(xyz, features, W1, b1r, g1r, beta1r,
                                 Wq, Wk, Wv)
    gidx, qks = _topk_call(x16, kf, q)
    gathered = _gather_call(tab.reshape(_M, _TW), gidx.reshape(_M * _K))
    g3 = gathered.reshape(_M, _K, _TW)
    res, attn = _attn_call(g3, q.reshape(_M, _DM), qks.reshape(_M, _K),
                           x16.reshape(_M, 16), features.reshape(_M, _DP),
                           Wd1p, bd1r, Wd2, bd2r, W2, b2r, g2r, beta2r)
    return res.reshape(_B, _N, _DP), attn.reshape(_B, _N, 1, _K)


# speedup vs baseline: 17.0741x; 1.0625x over previous
import functools

import jax
import jax.numpy as jnp
import numpy as np
from jax import lax
from jax.experimental import pallas as pl
from jax.experimental.pallas import tpu as pltpu
from jax.experimental.pallas import tpu_sc as plsc

_B, _N, _DP, _DM, _K = 4, 2048, 32, 64, 16
_M = _B * _N
_TN = 256
_TC = 256
_TW = 128
_EPS = 1e-5


def _ln(x, g, b):
    mu = jnp.mean(x, axis=-1, keepdims=True)
    var = jnp.mean((x - mu) ** 2, axis=-1, keepdims=True)
    return (x - mu) * lax.rsqrt(var + _EPS) * g + b


def _proj_body(xyz_ref, f_ref, W1_ref, b1_ref, g1_ref, beta1_ref,
               Wq_ref, Wk_ref, Wv_ref,
               q_ref, kf_ref, x16_ref, tab_ref):
    xyz3 = xyz_ref[0]
    f = f_ref[0]
    x = _ln(jnp.dot(f, W1_ref[...], preferred_element_type=jnp.float32)
            + b1_ref[...], g1_ref[...], beta1_ref[...])
    q_ref[0] = jnp.dot(x, Wq_ref[...], preferred_element_type=jnp.float32)
    kf_ref[0] = jnp.dot(x, Wk_ref[...], preferred_element_type=jnp.float32)
    vf = jnp.dot(x, Wv_ref[...], preferred_element_type=jnp.float32)
    sq = jnp.sum(xyz3 * xyz3, axis=-1, keepdims=True)
    x16 = jnp.concatenate(
        [xyz3, sq, jnp.zeros((_N, 12), jnp.float32)], axis=1)
    x16_ref[0] = x16
    tab_ref[0] = jnp.concatenate(
        [vf, x16, jnp.zeros((_N, _TW - _DM - 16), jnp.float32)], axis=1)


def _proj_call(xyz, features, W1, b1r, g1r, beta1r, Wq, Wk, Wv, interpret=False):
    full = lambda b: (b, 0, 0)
    w2d = lambda b: (0, 0)
    return pl.pallas_call(
        _proj_body,
        grid=(_B,),
        in_specs=[
            pl.BlockSpec((1, _N, 3), full),
            pl.BlockSpec((1, _N, _DP), full),
            pl.BlockSpec((_DP, _DM), w2d),
            pl.BlockSpec((1, _DM), w2d),
            pl.BlockSpec((1, _DM), w2d),
            pl.BlockSpec((1, _DM), w2d),
            pl.BlockSpec((_DM, _DM), w2d),
            pl.BlockSpec((_DM, _DM), w2d),
            pl.BlockSpec((_DM, _DM), w2d),
        ],
        out_specs=[
            pl.BlockSpec((1, _N, _DM), full),
            pl.BlockSpec((1, _N, _DM), full),
            pl.BlockSpec((1, _N, 16), full),
            pl.BlockSpec((1, _N, _TW), full),
        ],
        out_shape=[
            jax.ShapeDtypeStruct((_B, _N, _DM), jnp.float32),
            jax.ShapeDtypeStruct((_B, _N, _DM), jnp.float32),
            jax.ShapeDtypeStruct((_B, _N, 16), jnp.float32),
            jax.ShapeDtypeStruct((_B, _N, _TW), jnp.float32),
        ],
        compiler_params=pltpu.CompilerParams(
            dimension_semantics=("parallel",)),
        interpret=interpret,
    )(xyz, features, W1, b1r, g1r, beta1r, Wq, Wk, Wv)


def _topk_body(xt_ref, xa_ref, sqt_ref, kf_ref, q_ref, gidx_ref, qks_ref):
    b = pl.program_id(0)
    tile = xt_ref[0]
    xa = xa_ref[0]
    li = lax.broadcasted_iota(jnp.int32, (1, 16), 1)
    m3 = (li < 3).astype(jnp.float32)
    pd = lax.dot_general(tile * m3, xa * m3, (((1,), (1,)), ((), ())),
                         preferred_element_type=jnp.float32)
    dist = (tile[:, 3:4] + sqt_ref[0]) - 2.0 * pd
    qk = lax.dot_general(q_ref[0], kf_ref[0], (((1,), (1,)), ((), ())),
                         preferred_element_type=jnp.float32)

    iota_f = lax.broadcasted_iota(jnp.int32, (_TN, _N), 1).astype(jnp.float32)
    iota_k = lax.broadcasted_iota(jnp.int32, (_TN, _K), 1)
    idx_acc = jnp.zeros((_TN, _K), jnp.float32)
    qk_acc = jnp.zeros((_TN, _K), jnp.float32)
    d = dist
    for t in range(_K):
        m = jnp.min(d, axis=1, keepdims=True)
        cand = jnp.where(d <= m, iota_f, jnp.float32(_N))
        idx = jnp.min(cand, axis=1, keepdims=True)
        onehot = cand == idx
        qk_t = jnp.sum(jnp.where(onehot, qk, 0.0), axis=1, keepdims=True)
        d = jnp.where(onehot, jnp.float32(np.inf), d)
        sel = iota_k == t
        idx_acc = jnp.where(sel, idx, idx_acc)
        qk_acc = jnp.where(sel, qk_t, qk_acc)
    gidx_ref[0] = idx_acc.astype(jnp.int32) + b * _N
    qks_ref[0] = qk_acc


def _topk_call(x16, kf, q, interpret=False):
    sqt = x16[:, :, 3].reshape(_B, 1, _N)
    return pl.pallas_call(
        _topk_body,
        grid=(_B, _N // _TN),
        in_specs=[
            pl.BlockSpec((1, _TN, 16), lambda b, j: (b, j, 0)),
            pl.BlockSpec((1, _N, 16), lambda b, j: (b, 0, 0)),
            pl.BlockSpec((1, 1, _N), lambda b, j: (b, 0, 0)),
            pl.BlockSpec((1, _N, _DM), lambda b, j: (b, 0, 0)),
            pl.BlockSpec((1, _TN, _DM), lambda b, j: (b, j, 0)),
        ],
        out_specs=[
            pl.BlockSpec((1, _TN, _K), lambda b, j: (b, j, 0)),
            pl.BlockSpec((1, _TN, _K), lambda b, j: (b, j, 0)),
        ],
        out_shape=[
            jax.ShapeDtypeStruct((_B, _N, _K), jnp.int32),
            jax.ShapeDtypeStruct((_B, _N, _K), jnp.float32),
        ],
        compiler_params=pltpu.CompilerParams(
            dimension_semantics=("parallel", "parallel")),
        interpret=interpret,
    )(x16, x16, sqt, kf, q)


def _gather_call(tab, gidx):
    info = plsc.get_sparse_core_info()
    nw = info.num_cores * info.num_subcores
    tot = _M * _K
    per_w = tot // nw
    ch = 128
    n_ch = per_w // ch
    mesh = plsc.VectorSubcoreMesh(core_axis_name="c", subcore_axis_name="s")

    n_half = n_ch // 2

    @functools.partial(
        pl.kernel,
        mesh=mesh,
        out_type=jax.ShapeDtypeStruct((tot, _TW), jnp.float32),
        scratch_types=[
            pltpu.VMEM((ch,), jnp.int32),
            pltpu.VMEM((ch,), jnp.int32),
            pltpu.VMEM((ch, _TW), jnp.float32),
            pltpu.VMEM((ch, _TW), jnp.float32),
            pltpu.SemaphoreType.DMA,
            pltpu.SemaphoreType.DMA,
        ],
    )
    def _sc_gather(tab_hbm, idx_hbm, out_hbm, idx0, idx1, rows0, rows1,
                   sem0, sem1):
        wid = lax.axis_index("s") * info.num_cores + lax.axis_index("c")
        base = wid * per_w

        def issue(c, idx_v, rows_v, sem):
            pltpu.sync_copy(idx_hbm.at[pl.ds(base + c * ch, ch)], idx_v)
            pltpu.async_copy(tab_hbm.at[idx_v], rows_v, sem)

        def drain(c, idx_v, rows_v, sem):
            pltpu.make_async_copy(tab_hbm.at[idx_v], rows_v, sem).wait()
            pltpu.sync_copy(rows_v, out_hbm.at[pl.ds(base + c * ch, ch)])

        issue(0, idx0, rows0, sem0)

        def body(i, carry):
            c0 = 2 * i
            issue(c0 + 1, idx1, rows1, sem1)
            drain(c0, idx0, rows0, sem0)

            @pl.when(i + 1 < n_half)
            def _():
                issue(c0 + 2, idx0, rows0, sem0)

            drain(c0 + 1, idx1, rows1, sem1)
            return carry

        lax.fori_loop(0, n_half, body, 0)

    return _sc_gather(tab, gidx)


def _attn_body(g_ref, q_ref, qks_ref, x16_ref, f_ref,
               Wd1p_ref, bd1_ref, Wd2_ref, bd2_ref,
               W2_ref, b2_ref, g2_ref, beta2_ref,
               res_ref, attn_ref):
    g = g_ref[...]
    vsel = g[:, :, :_DM]
    xs = g[:, :, _DM:_DM + 16]
    xi = x16_ref[...]
    pos = xi[:, None, :] - xs
    posf = pos.reshape(_TC * _K, 16)
    h = jnp.maximum(
        jnp.dot(posf, Wd1p_ref[...], preferred_element_type=jnp.float32)
        + bd1_ref[...], 0.0)
    pe = (jnp.dot(h, Wd2_ref[...], preferred_element_type=jnp.float32)
          + bd2_ref[...]).reshape(_TC, _K, _DM)
    q = q_ref[...]
    s = (qks_ref[...] + jnp.sum(q[:, None, :] * pe, axis=-1)) \
        * jnp.float32(1.0 / np.sqrt(_DM))
    m = jnp.max(s, axis=-1, keepdims=True)
    e = jnp.exp(s - m)
    a = e / jnp.sum(e, axis=-1, keepdims=True)
    attn_ref[...] = a
    outv = jnp.sum(a[:, :, None] * (vsel + pe), axis=1)
    y = jnp.dot(outv, W2_ref[...], preferred_element_type=jnp.float32) \
        + b2_ref[...]
    res_ref[...] = _ln(y, g2_ref[...], beta2_ref[...]) + f_ref[...]


def _attn_call(g3, qf, qks, x16f, ff, Wd1p, bd1r, Wd2, bd2r, W2, b2r, g2r,
               beta2r, interpret=False):
    row = lambda i: (i, 0)
    row3 = lambda i: (i, 0, 0)
    w2d = lambda i: (0, 0)
    return pl.pallas_call(
        _attn_body,
        grid=(_M // _TC,),
        in_specs=[
            pl.BlockSpec((_TC, _K, _TW), row3),
            pl.BlockSpec((_TC, _DM), row),
            pl.BlockSpec((_TC, _K), row),
            pl.BlockSpec((_TC, 16), row),
            pl.BlockSpec((_TC, _DP), row),
            pl.BlockSpec((16, _DM), w2d),
            pl.BlockSpec((1, _DM), w2d),
            pl.BlockSpec((_DM, _DM), w2d),
            pl.BlockSpec((1, _DM), w2d),
            pl.BlockSpec((_DM, _DP), w2d),
            pl.BlockSpec((1, _DP), w2d),
            pl.BlockSpec((1, _DP), w2d),
            pl.BlockSpec((1, _DP), w2d),
        ],
        out_specs=[
            pl.BlockSpec((_TC, _DP), row),
            pl.BlockSpec((_TC, _K), row),
        ],
        out_shape=[
            jax.ShapeDtypeStruct((_M, _DP), jnp.float32),
            jax.ShapeDtypeStruct((_M, _K), jnp.float32),
        ],
        compiler_params=pltpu.CompilerParams(
            dimension_semantics=("parallel",)),
        interpret=interpret,
    )(g3, qf, qks, x16f, ff, Wd1p, bd1r, Wd2, bd2r, W2, b2r, g2r, beta2r)


def kernel(xyz, features, W1, b1, g1, beta1, Wd1, bd1, Wd2, bd2,
           Wq, Wk, Wv, W2, b2, g2, beta2):
    b1r = b1.reshape(1, _DM)
    g1r = g1.reshape(1, _DM)
    beta1r = beta1.reshape(1, _DM)
    bd1r = bd1.reshape(1, _DM)
    bd2r = bd2.reshape(1, _DM)
    b2r = b2.reshape(1, _DP)
    g2r = g2.reshape(1, _DP)
    beta2r = beta2.reshape(1, _DP)
    Wd1p = jnp.zeros((16, _DM), jnp.float32).at[:3].set(Wd1)

    q, kf, x16, tab = _proj_call(xyz, features, W1, b1r, g1r, beta1r,
                                 Wq, Wk, Wv)
    gidx, qks = _topk_call(x16, kf, q)
    gathered = _gather_call(tab.reshape(_M, _TW), gidx.reshape(_M * _K))
    g3 = gathered.reshape(_M, _K, _TW)
    res, attn = _attn_call(g3, q.reshape(_M, _DM), qks.reshape(_M, _K),
                           x16.reshape(_M, 16), features.reshape(_M, _DP),
                           Wd1p, bd1r, Wd2, bd2r, W2, b2r, g2r, beta2r)
    return res.reshape(_B, _N, _DP), attn.reshape(_B, _N, 1, _K)


# speedup vs baseline: 19.9613x; 1.1691x over previous
import functools

import jax
import jax.numpy as jnp
import numpy as np
from jax import lax
from jax.experimental import pallas as pl
from jax.experimental.pallas import tpu as pltpu
from jax.experimental.pallas import tpu_sc as plsc

_B, _N, _DP, _DM, _K = 4, 2048, 32, 64, 16
_M = _B * _N
_TN = 256
_TC = 256
_TW = 128
_EPS = 1e-5


def _ln(x, g, b):
    mu = jnp.mean(x, axis=-1, keepdims=True)
    var = jnp.mean((x - mu) ** 2, axis=-1, keepdims=True)
    return (x - mu) * lax.rsqrt(var + _EPS) * g + b


def _proj_body(xyz_ref, f_ref, W1_ref, b1_ref, g1_ref, beta1_ref,
               Wq_ref, Wk_ref, Wv_ref,
               q_ref, kf_ref, x16_ref, tab_ref):
    xyz3 = xyz_ref[0]
    f = f_ref[0]
    x = _ln(jnp.dot(f, W1_ref[...], preferred_element_type=jnp.float32)
            + b1_ref[...], g1_ref[...], beta1_ref[...])
    q_ref[0] = jnp.dot(x, Wq_ref[...], preferred_element_type=jnp.float32)
    kf_ref[0] = jnp.dot(x, Wk_ref[...], preferred_element_type=jnp.float32)
    vf = jnp.dot(x, Wv_ref[...], preferred_element_type=jnp.float32)
    sq = jnp.sum(xyz3 * xyz3, axis=-1, keepdims=True)
    x16 = jnp.concatenate(
        [xyz3, sq, jnp.zeros((_N, 12), jnp.float32)], axis=1)
    x16_ref[0] = x16
    tab_ref[0] = jnp.concatenate(
        [vf, x16, jnp.zeros((_N, _TW - _DM - 16), jnp.float32)], axis=1)


def _proj_call(xyz, features, W1, b1r, g1r, beta1r, Wq, Wk, Wv, interpret=False):
    full = lambda b: (b, 0, 0)
    w2d = lambda b: (0, 0)
    return pl.pallas_call(
        _proj_body,
        grid=(_B,),
        in_specs=[
            pl.BlockSpec((1, _N, 3), full),
            pl.BlockSpec((1, _N, _DP), full),
            pl.BlockSpec((_DP, _DM), w2d),
            pl.BlockSpec((1, _DM), w2d),
            pl.BlockSpec((1, _DM), w2d),
            pl.BlockSpec((1, _DM), w2d),
            pl.BlockSpec((_DM, _DM), w2d),
            pl.BlockSpec((_DM, _DM), w2d),
            pl.BlockSpec((_DM, _DM), w2d),
        ],
        out_specs=[
            pl.BlockSpec((1, _N, _DM), full),
            pl.BlockSpec((1, _N, _DM), full),
            pl.BlockSpec((1, _N, 16), full),
            pl.BlockSpec((1, _N, _TW), full),
        ],
        out_shape=[
            jax.ShapeDtypeStruct((_B, _N, _DM), jnp.float32),
            jax.ShapeDtypeStruct((_B, _N, _DM), jnp.float32),
            jax.ShapeDtypeStruct((_B, _N, 16), jnp.float32),
            jax.ShapeDtypeStruct((_B, _N, _TW), jnp.float32),
        ],
        compiler_params=pltpu.CompilerParams(
            dimension_semantics=("parallel",)),
        interpret=interpret,
    )(xyz, features, W1, b1r, g1r, beta1r, Wq, Wk, Wv)


def _topk_body(xt_ref, xa_ref, sqt_ref, kf_ref, q_ref, gidx_ref, qks_ref):
    b = pl.program_id(0)
    tile = xt_ref[0]
    xa = xa_ref[0]
    li = lax.broadcasted_iota(jnp.int32, (1, 16), 1)
    m3 = (li < 3).astype(jnp.float32)
    pd = lax.dot_general(tile * m3, xa * m3, (((1,), (1,)), ((), ())),
                         preferred_element_type=jnp.float32)
    dist = (tile[:, 3:4] + sqt_ref[0]) - 2.0 * pd
    qk = lax.dot_general(q_ref[0], kf_ref[0], (((1,), (1,)), ((), ())),
                         preferred_element_type=jnp.float32)

    iota_f = lax.broadcasted_iota(jnp.int32, (_TN, _N), 1).astype(jnp.float32)
    iota_k = lax.broadcasted_iota(jnp.int32, (_TN, _K), 1)
    idx_acc = jnp.zeros((_TN, _K), jnp.float32)
    qk_acc = jnp.zeros((_TN, _K), jnp.float32)
    d = dist
    for t in range(_K):
        m = jnp.min(d, axis=1, keepdims=True)
        cand = jnp.where(d <= m, iota_f, jnp.float32(_N))
        idx = jnp.min(cand, axis=1, keepdims=True)
        d = jnp.where(cand == idx, jnp.float32(np.inf), d)
        idx_acc = jnp.where(iota_k == t, idx, idx_acc)
    idx_i = idx_acc.astype(jnp.int32)
    idx_lo = jnp.bitwise_and(idx_i, 127)
    idx_hi = jnp.right_shift(idx_i, 7)
    for c in range(_N // 128):
        g = jnp.take_along_axis(qk[:, c * 128:(c + 1) * 128], idx_lo, axis=1)
        qk_acc = jnp.where(idx_hi == c, g, qk_acc)
    gidx_ref[0] = idx_i + b * _N
    qks_ref[0] = qk_acc


def _topk_call(x16, kf, q, interpret=False):
    sqt = x16[:, :, 3].reshape(_B, 1, _N)
    return pl.pallas_call(
        _topk_body,
        grid=(_B, _N // _TN),
        in_specs=[
            pl.BlockSpec((1, _TN, 16), lambda b, j: (b, j, 0)),
            pl.BlockSpec((1, _N, 16), lambda b, j: (b, 0, 0)),
            pl.BlockSpec((1, 1, _N), lambda b, j: (b, 0, 0)),
            pl.BlockSpec((1, _N, _DM), lambda b, j: (b, 0, 0)),
            pl.BlockSpec((1, _TN, _DM), lambda b, j: (b, j, 0)),
        ],
        out_specs=[
            pl.BlockSpec((1, _TN, _K), lambda b, j: (b, j, 0)),
            pl.BlockSpec((1, _TN, _K), lambda b, j: (b, j, 0)),
        ],
        out_shape=[
            jax.ShapeDtypeStruct((_B, _N, _K), jnp.int32),
            jax.ShapeDtypeStruct((_B, _N, _K), jnp.float32),
        ],
        compiler_params=pltpu.CompilerParams(
            dimension_semantics=("parallel", "parallel")),
        interpret=interpret,
    )(x16, x16, sqt, kf, q)


def _gather_call(tab, gidx):
    info = plsc.get_sparse_core_info()
    nw = info.num_cores * info.num_subcores
    tot = _M * _K
    per_w = tot // nw
    ch = 128
    n_ch = per_w // ch
    mesh = plsc.VectorSubcoreMesh(core_axis_name="c", subcore_axis_name="s")

    n_half = n_ch // 2

    @functools.partial(
        pl.kernel,
        mesh=mesh,
        out_type=jax.ShapeDtypeStruct((tot, _TW), jnp.float32),
        scratch_types=[
            pltpu.VMEM((ch,), jnp.int32),
            pltpu.VMEM((ch,), jnp.int32),
            pltpu.VMEM((ch, _TW), jnp.float32),
            pltpu.VMEM((ch, _TW), jnp.float32),
            pltpu.SemaphoreType.DMA,
            pltpu.SemaphoreType.DMA,
        ],
    )
    def _sc_gather(tab_hbm, idx_hbm, out_hbm, idx0, idx1, rows0, rows1,
                   sem0, sem1):
        wid = lax.axis_index("s") * info.num_cores + lax.axis_index("c")
        base = wid * per_w

        def issue(c, idx_v, rows_v, sem):
            pltpu.sync_copy(idx_hbm.at[pl.ds(base + c * ch, ch)], idx_v)
            pltpu.async_copy(tab_hbm.at[idx_v], rows_v, sem)

        def drain(c, idx_v, rows_v, sem):
            pltpu.make_async_copy(tab_hbm.at[idx_v], rows_v, sem).wait()
            pltpu.sync_copy(rows_v, out_hbm.at[pl.ds(base + c * ch, ch)])

        issue(0, idx0, rows0, sem0)

        def body(i, carry):
            c0 = 2 * i
            issue(c0 + 1, idx1, rows1, sem1)
            drain(c0, idx0, rows0, sem0)

            @pl.when(i + 1 < n_half)
            def _():
                issue(c0 + 2, idx0, rows0, sem0)

            drain(c0 + 1, idx1, rows1, sem1)
            return carry

        lax.fori_loop(0, n_half, body, 0)

    return _sc_gather(tab, gidx)


def _attn_body(g_ref, q_ref, qks_ref, x16_ref, f_ref,
               Wd1p_ref, bd1_ref, Wd2_ref, bd2_ref,
               W2_ref, b2_ref, g2_ref, beta2_ref,
               res_ref, attn_ref):
    g = g_ref[...]
    vsel = g[:, :, :_DM]
    xs = g[:, :, _DM:_DM + 16]
    xi = x16_ref[...]
    pos = xi[:, None, :] - xs
    posf = pos.reshape(_TC * _K, 16)
    h = jnp.maximum(
        jnp.dot(posf, Wd1p_ref[...], preferred_element_type=jnp.float32)
        + bd1_ref[...], 0.0)
    pe = (jnp.dot(h, Wd2_ref[...], preferred_element_type=jnp.float32)
          + bd2_ref[...]).reshape(_TC, _K, _DM)
    q = q_ref[...]
    s = (qks_ref[...] + jnp.sum(q[:, None, :] * pe, axis=-1)) \
        * jnp.float32(1.0 / np.sqrt(_DM))
    m = jnp.max(s, axis=-1, keepdims=True)
    e = jnp.exp(s - m)
    a = e / jnp.sum(e, axis=-1, keepdims=True)
    attn_ref[...] = a
    outv = jnp.sum(a[:, :, None] * (vsel + pe), axis=1)
    y = jnp.dot(outv, W2_ref[...], preferred_element_type=jnp.float32) \
        + b2_ref[...]
    res_ref[...] = _ln(y, g2_ref[...], beta2_ref[...]) + f_ref[...]


def _attn_call(g3, qf, qks, x16f, ff, Wd1p, bd1r, Wd2, bd2r, W2, b2r, g2r,
               beta2r, interpret=False):
    row = lambda i: (i, 0)
    row3 = lambda i: (i, 0, 0)
    w2d = lambda i: (0, 0)
    return pl.pallas_call(
        _attn_body,
        grid=(_M // _TC,),
        in_specs=[
            pl.BlockSpec((_TC, _K, _TW), row3),
            pl.BlockSpec((_TC, _DM), row),
            pl.BlockSpec((_TC, _K), row),
            pl.BlockSpec((_TC, 16), row),
            pl.BlockSpec((_TC, _DP), row),
            pl.BlockSpec((16, _DM), w2d),
            pl.BlockSpec((1, _DM), w2d),
            pl.BlockSpec((_DM, _DM), w2d),
            pl.BlockSpec((1, _DM), w2d),
            pl.BlockSpec((_DM, _DP), w2d),
            pl.BlockSpec((1, _DP), w2d),
            pl.BlockSpec((1, _DP), w2d),
            pl.BlockSpec((1, _DP), w2d),
        ],
        out_specs=[
            pl.BlockSpec((_TC, _DP), row),
            pl.BlockSpec((_TC, _K), row),
        ],
        out_shape=[
            jax.ShapeDtypeStruct((_M, _DP), jnp.float32),
            jax.ShapeDtypeStruct((_M, _K), jnp.float32),
        ],
        compiler_params=pltpu.CompilerParams(
            dimension_semantics=("parallel",)),
        interpret=interpret,
    )(g3, qf, qks, x16f, ff, Wd1p, bd1r, Wd2, bd2r, W2, b2r, g2r, beta2r)


def kernel(xyz, features, W1, b1, g1, beta1, Wd1, bd1, Wd2, bd2,
           Wq, Wk, Wv, W2, b2, g2, beta2):
    b1r = b1.reshape(1, _DM)
    g1r = g1.reshape(1, _DM)
    beta1r = beta1.reshape(1, _DM)
    bd1r = bd1.reshape(1, _DM)
    bd2r = bd2.reshape(1, _DM)
    b2r = b2.reshape(1, _DP)
    g2r = g2.reshape(1, _DP)
    beta2r = beta2.reshape(1, _DP)
    Wd1p = jnp.zeros((16, _DM), jnp.float32).at[:3].set(Wd1)

    q, kf, x16, tab = _proj_call(xyz, features, W1, b1r, g1r, beta1r,
                                 Wq, Wk, Wv)
    gidx, qks = _topk_call(x16, kf, q)
    gathered = _gather_call(tab.reshape(_M, _TW), gidx.reshape(_M * _K))
    g3 = gathered.reshape(_M, _K, _TW)
    res, attn = _attn_call(g3, q.reshape(_M, _DM), qks.reshape(_M, _K),
                           x16.reshape(_M, 16), features.reshape(_M, _DP),
                           Wd1p, bd1r, Wd2, bd2r, W2, b2r, g2r, beta2r)
    return res.reshape(_B, _N, _DP), attn.reshape(_B, _N, 1, _K)


# speedup vs baseline: 20.0827x; 1.0061x over previous
import functools

import jax
import jax.numpy as jnp
import numpy as np
from jax import lax
from jax.experimental import pallas as pl
from jax.experimental.pallas import tpu as pltpu
from jax.experimental.pallas import tpu_sc as plsc

_B, _N, _DP, _DM, _K = 4, 2048, 32, 64, 16
_M = _B * _N
_TN = 256
_TC = 256
_TW = 128
_EPS = 1e-5


def _ln(x, g, b):
    mu = jnp.mean(x, axis=-1, keepdims=True)
    var = jnp.mean((x - mu) ** 2, axis=-1, keepdims=True)
    return (x - mu) * lax.rsqrt(var + _EPS) * g + b


def _proj_body(xyz_ref, f_ref, W1_ref, b1_ref, g1_ref, beta1_ref,
               Wq_ref, Wk_ref, Wv_ref,
               q_ref, kf_ref, x16_ref, tab_ref):
    xyz3 = xyz_ref[0]
    f = f_ref[0]
    x = _ln(jnp.dot(f, W1_ref[...], preferred_element_type=jnp.float32)
            + b1_ref[...], g1_ref[...], beta1_ref[...])
    q_ref[0] = jnp.dot(x, Wq_ref[...], preferred_element_type=jnp.float32)
    kf_ref[0] = jnp.dot(x, Wk_ref[...], preferred_element_type=jnp.float32)
    vf = jnp.dot(x, Wv_ref[...], preferred_element_type=jnp.float32)
    sq = jnp.sum(xyz3 * xyz3, axis=-1, keepdims=True)
    x16 = jnp.concatenate(
        [xyz3, sq, jnp.zeros((_N, 12), jnp.float32)], axis=1)
    x16_ref[0] = x16
    tab_ref[0] = jnp.concatenate(
        [vf, x16, jnp.zeros((_N, _TW - _DM - 16), jnp.float32)], axis=1)


def _proj_call(xyz, features, W1, b1r, g1r, beta1r, Wq, Wk, Wv, interpret=False):
    full = lambda b: (b, 0, 0)
    w2d = lambda b: (0, 0)
    return pl.pallas_call(
        _proj_body,
        grid=(_B,),
        in_specs=[
            pl.BlockSpec((1, _N, 3), full),
            pl.BlockSpec((1, _N, _DP), full),
            pl.BlockSpec((_DP, _DM), w2d),
            pl.BlockSpec((1, _DM), w2d),
            pl.BlockSpec((1, _DM), w2d),
            pl.BlockSpec((1, _DM), w2d),
            pl.BlockSpec((_DM, _DM), w2d),
            pl.BlockSpec((_DM, _DM), w2d),
            pl.BlockSpec((_DM, _DM), w2d),
        ],
        out_specs=[
            pl.BlockSpec((1, _N, _DM), full),
            pl.BlockSpec((1, _N, _DM), full),
            pl.BlockSpec((1, _N, 16), full),
            pl.BlockSpec((1, _N, _TW), full),
        ],
        out_shape=[
            jax.ShapeDtypeStruct((_B, _N, _DM), jnp.float32),
            jax.ShapeDtypeStruct((_B, _N, _DM), jnp.float32),
            jax.ShapeDtypeStruct((_B, _N, 16), jnp.float32),
            jax.ShapeDtypeStruct((_B, _N, _TW), jnp.float32),
        ],
        compiler_params=pltpu.CompilerParams(
            dimension_semantics=("parallel",)),
        interpret=interpret,
    )(xyz, features, W1, b1r, g1r, beta1r, Wq, Wk, Wv)


def _topk_body(xt_ref, xa_ref, sqt_ref, kf_ref, q_ref, gidx_ref, qks_ref):
    b = pl.program_id(0)
    tile = xt_ref[0]
    xa = xa_ref[0]
    li = lax.broadcasted_iota(jnp.int32, (1, 16), 1)
    m3 = (li < 3).astype(jnp.float32)
    pd = lax.dot_general(tile * m3, xa * m3, (((1,), (1,)), ((), ())),
                         preferred_element_type=jnp.float32)
    dist = (tile[:, 3:4] + sqt_ref[0]) - 2.0 * pd
    qk = lax.dot_general(q_ref[0], kf_ref[0], (((1,), (1,)), ((), ())),
                         preferred_element_type=jnp.float32)

    iota_f = lax.broadcasted_iota(jnp.int32, (_TN, _N), 1).astype(jnp.float32)
    iota_k = lax.broadcasted_iota(jnp.int32, (_TN, _K), 1)
    idx_acc = jnp.zeros((_TN, _K), jnp.float32)
    qk_acc = jnp.zeros((_TN, _K), jnp.float32)
    d = dist
    for t in range(_K):
        m = jnp.min(d, axis=1, keepdims=True)
        cand = jnp.where(d <= m, iota_f, jnp.float32(_N))
        idx = jnp.min(cand, axis=1, keepdims=True)
        d = jnp.where(cand == idx, jnp.float32(np.inf), d)
        idx_acc = jnp.where(iota_k == t, idx, idx_acc)
    idx_i = idx_acc.astype(jnp.int32)
    idx_lo = jnp.bitwise_and(idx_i, 127)
    idx_hi = jnp.right_shift(idx_i, 7)
    for c in range(_N // 128):
        g = jnp.take_along_axis(qk[:, c * 128:(c + 1) * 128], idx_lo, axis=1)
        qk_acc = jnp.where(idx_hi == c, g, qk_acc)
    gidx_ref[0] = idx_i + b * _N
    qks_ref[0] = qk_acc


def _topk_call(x16, kf, q, interpret=False):
    nb = x16.shape[0]
    sqt = x16[:, :, 3].reshape(nb, 1, _N)
    return pl.pallas_call(
        _topk_body,
        grid=(nb, _N // _TN),
        in_specs=[
            pl.BlockSpec((1, _TN, 16), lambda b, j: (b, j, 0)),
            pl.BlockSpec((1, _N, 16), lambda b, j: (b, 0, 0)),
            pl.BlockSpec((1, 1, _N), lambda b, j: (b, 0, 0)),
            pl.BlockSpec((1, _N, _DM), lambda b, j: (b, 0, 0)),
            pl.BlockSpec((1, _TN, _DM), lambda b, j: (b, j, 0)),
        ],
        out_specs=[
            pl.BlockSpec((1, _TN, _K), lambda b, j: (b, j, 0)),
            pl.BlockSpec((1, _TN, _K), lambda b, j: (b, j, 0)),
        ],
        out_shape=[
            jax.ShapeDtypeStruct((nb, _N, _K), jnp.int32),
            jax.ShapeDtypeStruct((nb, _N, _K), jnp.float32),
        ],
        compiler_params=pltpu.CompilerParams(
            dimension_semantics=("parallel", "parallel")),
        interpret=interpret,
    )(x16, x16, sqt, kf, q)


def _gather_call(tab, gidx):
    info = plsc.get_sparse_core_info()
    nw = info.num_cores * info.num_subcores
    tot = gidx.shape[0]
    per_w = tot // nw
    ch = 128
    n_ch = per_w // ch
    mesh = plsc.VectorSubcoreMesh(core_axis_name="c", subcore_axis_name="s")

    n_half = n_ch // 2

    @functools.partial(
        pl.kernel,
        mesh=mesh,
        out_type=jax.ShapeDtypeStruct((tot, _TW), jnp.float32),
        scratch_types=[
            pltpu.VMEM((ch,), jnp.int32),
            pltpu.VMEM((ch,), jnp.int32),
            pltpu.VMEM((ch, _TW), jnp.float32),
            pltpu.VMEM((ch, _TW), jnp.float32),
            pltpu.SemaphoreType.DMA,
            pltpu.SemaphoreType.DMA,
        ],
    )
    def _sc_gather(tab_hbm, idx_hbm, out_hbm, idx0, idx1, rows0, rows1,
                   sem0, sem1):
        wid = lax.axis_index("s") * info.num_cores + lax.axis_index("c")
        base = wid * per_w

        def issue(c, idx_v, rows_v, sem):
            pltpu.sync_copy(idx_hbm.at[pl.ds(base + c * ch, ch)], idx_v)
            pltpu.async_copy(tab_hbm.at[idx_v], rows_v, sem)

        def drain(c, idx_v, rows_v, sem):
            pltpu.make_async_copy(tab_hbm.at[idx_v], rows_v, sem).wait()
            pltpu.sync_copy(rows_v, out_hbm.at[pl.ds(base + c * ch, ch)])

        issue(0, idx0, rows0, sem0)

        def body(i, carry):
            c0 = 2 * i
            issue(c0 + 1, idx1, rows1, sem1)
            drain(c0, idx0, rows0, sem0)

            @pl.when(i + 1 < n_half)
            def _():
                issue(c0 + 2, idx0, rows0, sem0)

            drain(c0 + 1, idx1, rows1, sem1)
            return carry

        lax.fori_loop(0, n_half, body, 0)

    return _sc_gather(tab, gidx)


def _attn_body(g_ref, q_ref, qks_ref, x16_ref, f_ref,
               Wd1p_ref, bd1_ref, Wd2_ref, bd2_ref,
               W2_ref, b2_ref, g2_ref, beta2_ref,
               res_ref, attn_ref):
    g = g_ref[...]
    vsel = g[:, :, :_DM]
    xs = g[:, :, _DM:_DM + 16]
    xi = x16_ref[...]
    pos = xi[:, None, :] - xs
    posf = pos.reshape(_TC * _K, 16)
    h = jnp.maximum(
        jnp.dot(posf, Wd1p_ref[...], preferred_element_type=jnp.float32)
        + bd1_ref[...], 0.0)
    pe = (jnp.dot(h, Wd2_ref[...], preferred_element_type=jnp.float32)
          + bd2_ref[...]).reshape(_TC, _K, _DM)
    q = q_ref[...]
    s = (qks_ref[...] + jnp.sum(q[:, None, :] * pe, axis=-1)) \
        * jnp.float32(1.0 / np.sqrt(_DM))
    m = jnp.max(s, axis=-1, keepdims=True)
    e = jnp.exp(s - m)
    a = e / jnp.sum(e, axis=-1, keepdims=True)
    attn_ref[...] = a
    outv = jnp.sum(a[:, :, None] * (vsel + pe), axis=1)
    y = jnp.dot(outv, W2_ref[...], preferred_element_type=jnp.float32) \
        + b2_ref[...]
    res_ref[...] = _ln(y, g2_ref[...], beta2_ref[...]) + f_ref[...]


def _attn_call(g3, qf, qks, x16f, ff, Wd1p, bd1r, Wd2, bd2r, W2, b2r, g2r,
               beta2r, interpret=False):
    row = lambda i: (i, 0)
    row3 = lambda i: (i, 0, 0)
    w2d = lambda i: (0, 0)
    m = qf.shape[0]
    return pl.pallas_call(
        _attn_body,
        grid=(m // _TC,),
        in_specs=[
            pl.BlockSpec((_TC, _K, _TW), row3),
            pl.BlockSpec((_TC, _DM), row),
            pl.BlockSpec((_TC, _K), row),
            pl.BlockSpec((_TC, 16), row),
            pl.BlockSpec((_TC, _DP), row),
            pl.BlockSpec((16, _DM), w2d),
            pl.BlockSpec((1, _DM), w2d),
            pl.BlockSpec((_DM, _DM), w2d),
            pl.BlockSpec((1, _DM), w2d),
            pl.BlockSpec((_DM, _DP), w2d),
            pl.BlockSpec((1, _DP), w2d),
            pl.BlockSpec((1, _DP), w2d),
            pl.BlockSpec((1, _DP), w2d),
        ],
        out_specs=[
            pl.BlockSpec((_TC, _DP), row),
            pl.BlockSpec((_TC, _K), row),
        ],
        out_shape=[
            jax.ShapeDtypeStruct((m, _DP), jnp.float32),
            jax.ShapeDtypeStruct((m, _K), jnp.float32),
        ],
        compiler_params=pltpu.CompilerParams(
            dimension_semantics=("parallel",)),
        interpret=interpret,
    )(g3, qf, qks, x16f, ff, Wd1p, bd1r, Wd2, bd2r, W2, b2r, g2r, beta2r)


def kernel(xyz, features, W1, b1, g1, beta1, Wd1, bd1, Wd2, bd2,
           Wq, Wk, Wv, W2, b2, g2, beta2):
    b1r = b1.reshape(1, _DM)
    g1r = g1.reshape(1, _DM)
    beta1r = beta1.reshape(1, _DM)
    bd1r = bd1.reshape(1, _DM)
    bd2r = bd2.reshape(1, _DM)
    b2r = b2.reshape(1, _DP)
    g2r = g2.reshape(1, _DP)
    beta2r = beta2.reshape(1, _DP)
    Wd1p = jnp.zeros((16, _DM), jnp.float32).at[:3].set(Wd1)

    q, kf, x16, tab = _proj_call(xyz, features, W1, b1r, g1r, beta1r,
                                 Wq, Wk, Wv)
    res_l, attn_l = [], []
    for b in range(_B):
        gidx_b, qks_b = _topk_call(x16[b:b + 1], kf[b:b + 1], q[b:b + 1])
        gath_b = _gather_call(tab[b], gidx_b.reshape(_N * _K))
        res_b, attn_b = _attn_call(
            gath_b.reshape(_N, _K, _TW), q[b], qks_b.reshape(_N, _K),
            x16[b], features[b], Wd1p, bd1r, Wd2, bd2r, W2, b2r, g2r, beta2r)
        res_l.append(res_b)
        attn_l.append(attn_b)
    return (jnp.stack(res_l), jnp.stack(attn_l).reshape(_B, _N, 1, _K))


# speedup vs baseline: 21.3041x; 1.0608x over previous
import functools

import jax
import jax.numpy as jnp
import numpy as np
from jax import lax
from jax.experimental import pallas as pl
from jax.experimental.pallas import tpu as pltpu
from jax.experimental.pallas import tpu_sc as plsc

_B, _N, _DP, _DM, _K = 4, 2048, 32, 64, 16
_M = _B * _N
_TN = 256
_TC = 512
_TW = 128
_EPS = 1e-5


def _ln(x, g, b):
    mu = jnp.mean(x, axis=-1, keepdims=True)
    var = jnp.mean((x - mu) ** 2, axis=-1, keepdims=True)
    return (x - mu) * lax.rsqrt(var + _EPS) * g + b


def _proj_body(xyz_ref, f_ref, W1_ref, b1_ref, g1_ref, beta1_ref,
               Wq_ref, Wk_ref, Wv_ref,
               q_ref, kf_ref, x16_ref, sqt_ref, tab_ref):
    xyz3 = xyz_ref[0]
    f = f_ref[0]
    x = _ln(jnp.dot(f, W1_ref[...], preferred_element_type=jnp.float32)
            + b1_ref[...], g1_ref[...], beta1_ref[...])
    q_ref[0] = jnp.dot(x, Wq_ref[...], preferred_element_type=jnp.float32)
    kf_ref[0] = jnp.dot(x, Wk_ref[...], preferred_element_type=jnp.float32)
    vf = jnp.dot(x, Wv_ref[...], preferred_element_type=jnp.float32)
    sq = jnp.sum(xyz3 * xyz3, axis=-1, keepdims=True)
    x16 = jnp.concatenate(
        [xyz3, sq, jnp.zeros((_N, 12), jnp.float32)], axis=1)
    x16_ref[0] = x16
    sqt_ref[0] = sq.reshape(1, _N)
    tab_ref[0] = jnp.concatenate(
        [vf, x16, jnp.zeros((_N, _TW - _DM - 16), jnp.float32)], axis=1)


def _proj_call(xyz, features, W1, b1r, g1r, beta1r, Wq, Wk, Wv, interpret=False):
    full = lambda b: (b, 0, 0)
    w2d = lambda b: (0, 0)
    return pl.pallas_call(
        _proj_body,
        grid=(_B,),
        in_specs=[
            pl.BlockSpec((1, _N, 3), full),
            pl.BlockSpec((1, _N, _DP), full),
            pl.BlockSpec((_DP, _DM), w2d),
            pl.BlockSpec((1, _DM), w2d),
            pl.BlockSpec((1, _DM), w2d),
            pl.BlockSpec((1, _DM), w2d),
            pl.BlockSpec((_DM, _DM), w2d),
            pl.BlockSpec((_DM, _DM), w2d),
            pl.BlockSpec((_DM, _DM), w2d),
        ],
        out_specs=[
            pl.BlockSpec((1, _N, _DM), full),
            pl.BlockSpec((1, _N, _DM), full),
            pl.BlockSpec((1, _N, 16), full),
            pl.BlockSpec((1, 1, _N), full),
            pl.BlockSpec((1, _N, _TW), full),
        ],
        out_shape=[
            jax.ShapeDtypeStruct((_B, _N, _DM), jnp.float32),
            jax.ShapeDtypeStruct((_B, _N, _DM), jnp.float32),
            jax.ShapeDtypeStruct((_B, _N, 16), jnp.float32),
            jax.ShapeDtypeStruct((_B, 1, _N), jnp.float32),
            jax.ShapeDtypeStruct((_B, _N, _TW), jnp.float32),
        ],
        compiler_params=pltpu.CompilerParams(
            dimension_semantics=("parallel",)),
        interpret=interpret,
    )(xyz, features, W1, b1r, g1r, beta1r, Wq, Wk, Wv)


def _topk_body(xt_ref, xa_ref, sqt_ref, kf_ref, q_ref, gidx_ref, qks_ref):
    tile = xt_ref[0]
    xa = xa_ref[0]
    li = lax.broadcasted_iota(jnp.int32, (1, 16), 1)
    m3 = (li < 3).astype(jnp.float32)
    pd = lax.dot_general(tile * m3, xa * m3, (((1,), (1,)), ((), ())),
                         preferred_element_type=jnp.float32)
    dist = (tile[:, 3:4] + sqt_ref[0]) - 2.0 * pd
    qk = lax.dot_general(q_ref[0], kf_ref[0], (((1,), (1,)), ((), ())),
                         preferred_element_type=jnp.float32)

    iota_f = lax.broadcasted_iota(jnp.int32, (_TN, _N), 1).astype(jnp.float32)
    iota_k = lax.broadcasted_iota(jnp.int32, (_TN, _K), 1)
    idx_acc = jnp.zeros((_TN, _K), jnp.float32)
    qk_acc = jnp.zeros((_TN, _K), jnp.float32)
    d = dist
    for t in range(_K):
        m = jnp.min(d, axis=1, keepdims=True)
        cand = jnp.where(d == m, iota_f, jnp.float32(_N))
        idx = jnp.min(cand, axis=1, keepdims=True)
        d = jnp.where(cand == idx, jnp.float32(np.inf), d)
        idx_acc = jnp.where(iota_k == t, idx, idx_acc)
    idx_i = idx_acc.astype(jnp.int32)
    idx_lo = jnp.bitwise_and(idx_i, 127)
    idx_hi = jnp.right_shift(idx_i, 7)
    for c in range(_N // 128):
        g = jnp.take_along_axis(qk[:, c * 128:(c + 1) * 128], idx_lo, axis=1)
        qk_acc = jnp.where(idx_hi == c, g, qk_acc)
    gidx_ref[...] = idx_i
    qks_ref[...] = qk_acc


def _topk_call(x16, sqt, kf, q, b, interpret=False):
    return pl.pallas_call(
        _topk_body,
        grid=(_N // _TN,),
        in_specs=[
            pl.BlockSpec((1, _TN, 16), lambda j: (b, j, 0)),
            pl.BlockSpec((1, _N, 16), lambda j: (b, 0, 0)),
            pl.BlockSpec((1, 1, _N), lambda j: (b, 0, 0)),
            pl.BlockSpec((1, _N, _DM), lambda j: (b, 0, 0)),
            pl.BlockSpec((1, _TN, _DM), lambda j: (b, j, 0)),
        ],
        out_specs=[
            pl.BlockSpec((_TN, _K), lambda j: (j, 0)),
            pl.BlockSpec((_TN, _K), lambda j: (j, 0)),
        ],
        out_shape=[
            jax.ShapeDtypeStruct((_N, _K), jnp.int32),
            jax.ShapeDtypeStruct((_N, _K), jnp.float32),
        ],
        compiler_params=pltpu.CompilerParams(
            dimension_semantics=("parallel",)),
        interpret=interpret,
    )(x16, x16, sqt, kf, q)


def _gather_call(tab, gidx):
    info = plsc.get_sparse_core_info()
    nw = info.num_cores * info.num_subcores
    tot = gidx.shape[0]
    per_w = tot // nw
    ch = 128
    n_ch = per_w // ch
    mesh = plsc.VectorSubcoreMesh(core_axis_name="c", subcore_axis_name="s")

    n_half = n_ch // 2

    @functools.partial(
        pl.kernel,
        mesh=mesh,
        out_type=jax.ShapeDtypeStruct((tot, _TW), jnp.float32),
        scratch_types=[
            pltpu.VMEM((ch,), jnp.int32),
            pltpu.VMEM((ch,), jnp.int32),
            pltpu.VMEM((ch, _TW), jnp.float32),
            pltpu.VMEM((ch, _TW), jnp.float32),
            pltpu.SemaphoreType.DMA,
            pltpu.SemaphoreType.DMA,
        ],
    )
    def _sc_gather(tab_hbm, idx_hbm, out_hbm, idx0, idx1, rows0, rows1,
                   sem0, sem1):
        wid = lax.axis_index("s") * info.num_cores + lax.axis_index("c")
        base = wid * per_w

        def issue(c, idx_v, rows_v, sem):
            pltpu.sync_copy(idx_hbm.at[pl.ds(base + c * ch, ch)], idx_v)
            pltpu.async_copy(tab_hbm.at[idx_v], rows_v, sem)

        def drain(c, idx_v, rows_v, sem):
            pltpu.make_async_copy(tab_hbm.at[idx_v], rows_v, sem).wait()
            pltpu.sync_copy(rows_v, out_hbm.at[pl.ds(base + c * ch, ch)])

        issue(0, idx0, rows0, sem0)

        def body(i, carry):
            c0 = 2 * i
            issue(c0 + 1, idx1, rows1, sem1)
            drain(c0, idx0, rows0, sem0)

            @pl.when(i + 1 < n_half)
            def _():
                issue(c0 + 2, idx0, rows0, sem0)

            drain(c0 + 1, idx1, rows1, sem1)
            return carry

        lax.fori_loop(0, n_half, body, 0)

    return _sc_gather(tab, gidx)


def _attn_body(g_ref, q_ref, qks_ref, x16_ref, f_ref,
               Wd1p_ref, bd1_ref, Wd2_ref, bd2_ref,
               W2_ref, b2_ref, g2_ref, beta2_ref,
               res_ref, attn_ref):
    g = g_ref[...]
    vsel = g[:, :, :_DM]
    xs = g[:, :, _DM:_DM + 16]
    xi = x16_ref[0]
    pos = xi[:, None, :] - xs
    posf = pos.reshape(_TC * _K, 16)
    h = jnp.maximum(
        jnp.dot(posf, Wd1p_ref[...], preferred_element_type=jnp.float32)
        + bd1_ref[...], 0.0)
    pe = (jnp.dot(h, Wd2_ref[...], preferred_element_type=jnp.float32)
          + bd2_ref[...]).reshape(_TC, _K, _DM)
    q = q_ref[0]
    s = (qks_ref[...] + jnp.sum(q[:, None, :] * pe, axis=-1)) \
        * jnp.float32(1.0 / np.sqrt(_DM))
    m = jnp.max(s, axis=-1, keepdims=True)
    e = jnp.exp(s - m)
    a = e / jnp.sum(e, axis=-1, keepdims=True)
    attn_ref[...] = a
    outv = jnp.sum(a[:, :, None] * (vsel + pe), axis=1)
    y = jnp.dot(outv, W2_ref[...], preferred_element_type=jnp.float32) \
        + b2_ref[...]
    res_ref[...] = _ln(y, g2_ref[...], beta2_ref[...]) + f_ref[0]


def _attn_call(g3, qf, qks, x16f, ff, Wd1p, bd1r, Wd2, bd2r, W2, b2r, g2r,
               beta2r, b, interpret=False):
    row = lambda i: (i, 0)
    rowb = lambda i: (b, i, 0)
    row3 = lambda i: (i, 0, 0)
    w2d = lambda i: (0, 0)
    return pl.pallas_call(
        _attn_body,
        grid=(_N // _TC,),
        in_specs=[
            pl.BlockSpec((_TC, _K, _TW), row3),
            pl.BlockSpec((1, _TC, _DM), rowb),
            pl.BlockSpec((_TC, _K), row),
            pl.BlockSpec((1, _TC, 16), rowb),
            pl.BlockSpec((1, _TC, _DP), rowb),
            pl.BlockSpec((16, _DM), w2d),
            pl.BlockSpec((1, _DM), w2d),
            pl.BlockSpec((_DM, _DM), w2d),
            pl.BlockSpec((1, _DM), w2d),
            pl.BlockSpec((_DM, _DP), w2d),
            pl.BlockSpec((1, _DP), w2d),
            pl.BlockSpec((1, _DP), w2d),
            pl.BlockSpec((1, _DP), w2d),
        ],
        out_specs=[
            pl.BlockSpec((_TC, _DP), row),
            pl.BlockSpec((_TC, _K), row),
        ],
        out_shape=[
            jax.ShapeDtypeStruct((_N, _DP), jnp.float32),
            jax.ShapeDtypeStruct((_N, _K), jnp.float32),
        ],
        compiler_params=pltpu.CompilerParams(
            dimension_semantics=("parallel",)),
        interpret=interpret,
    )(g3, qf, qks, x16f, ff, Wd1p, bd1r, Wd2, bd2r, W2, b2r, g2r, beta2r)


def kernel(xyz, features, W1, b1, g1, beta1, Wd1, bd1, Wd2, bd2,
           Wq, Wk, Wv, W2, b2, g2, beta2):
    b1r = b1.reshape(1, _DM)
    g1r = g1.reshape(1, _DM)
    beta1r = beta1.reshape(1, _DM)
    bd1r = bd1.reshape(1, _DM)
    bd2r = bd2.reshape(1, _DM)
    b2r = b2.reshape(1, _DP)
    g2r = g2.reshape(1, _DP)
    beta2r = beta2.reshape(1, _DP)
    Wd1p = jnp.zeros((16, _DM), jnp.float32).at[:3].set(Wd1)

    q, kf, x16, sqt, tab = _proj_call(xyz, features, W1, b1r, g1r, beta1r,
                                      Wq, Wk, Wv)
    res_l, attn_l = [], []
    for b in range(_B):
        gidx_b, qks_b = _topk_call(x16, sqt, kf, q, b)
        gath_b = _gather_call(tab[b], gidx_b.reshape(_N * _K))
        res_b, attn_b = _attn_call(
            gath_b.reshape(_N, _K, _TW), q, qks_b, x16, features,
            Wd1p, bd1r, Wd2, bd2r, W2, b2r, g2r, beta2r, b)
        res_l.append(res_b)
        attn_l.append(attn_b)
    return (jnp.stack(res_l), jnp.stack(attn_l).reshape(_B, _N, 1, _K))
